# speedup vs baseline: 26.0400x; 26.0400x over previous
import functools
import jax
import jax.numpy as jnp
from jax import lax
from jax.experimental import pallas as pl
from jax.experimental.pallas import tpu as pltpu

B, C, D, K = 16, 256, 512, 16


def _tc_body(idx_ref, sc_ref, h_ref, WvT_ref, WoT_ref, Wg1T_ref, Wg2T_ref,
             bv_ref, bo_ref, bg_ref, out_ref):
    idx = idx_ref[0]
    s = sc_ref[0]
    m = jnp.max(s, axis=-1, keepdims=True)
    e = jnp.exp(s - m)
    attn = e / jnp.sum(e, axis=-1, keepdims=True)

    iota = lax.broadcasted_iota(jnp.int32, (C, C), 1)
    a = jnp.zeros((C, C), jnp.float32)
    for k in range(K):
        a = a + jnp.where(idx[:, k][:, None] == iota,
                          attn[:, k][:, None], 0.0)

    hb = h_ref[0]
    agg0 = jnp.dot(a, hb, preferred_element_type=jnp.float32)
    agg = jnp.dot(agg0, WvT_ref[...], preferred_element_type=jnp.float32) + bv_ref[...]
    g = jax.nn.sigmoid(
        jnp.dot(hb, Wg1T_ref[...], preferred_element_type=jnp.float32)
        + jnp.dot(agg, Wg2T_ref[...], preferred_element_type=jnp.float32)
        + bg_ref[...])
    out_ref[0] = g * (jnp.dot(agg, WoT_ref[...], preferred_element_type=jnp.float32)
                      + bo_ref[...])


def kernel(h, topk_idx, topk_scores, Wv, bv, Wo, bo, Wg, bg):
    idx32 = topk_idx.astype(jnp.int32)
    WvT = Wv.T
    WoT = Wo.T
    Wg1T = Wg[:, :D].T
    Wg2T = Wg[:, D:].T

    out = pl.pallas_call(
        _tc_body,
        grid=(B,),
        in_specs=[
            pl.BlockSpec((1, C, K), lambda b: (b, 0, 0)),
            pl.BlockSpec((1, C, K), lambda b: (b, 0, 0)),
            pl.BlockSpec((1, C, D), lambda b: (b, 0, 0)),
            pl.BlockSpec((D, D), lambda b: (0, 0)),
            pl.BlockSpec((D, D), lambda b: (0, 0)),
            pl.BlockSpec((D, D), lambda b: (0, 0)),
            pl.BlockSpec((D, D), lambda b: (0, 0)),
            pl.BlockSpec((1, D), lambda b: (0, 0)),
            pl.BlockSpec((1, D), lambda b: (0, 0)),
            pl.BlockSpec((1, D), lambda b: (0, 0)),
        ],
        out_specs=pl.BlockSpec((1, C, D), lambda b: (b, 0, 0)),
        out_shape=jax.ShapeDtypeStruct((B, C, D), jnp.float32),
    )(idx32, topk_scores, h, WvT, WoT, Wg1T, Wg2T,
      bv[None, :], bo[None, :], bg[None, :])
    return out
